# baseline (speedup 1.0000x reference)
import functools

import jax
import jax.numpy as jnp
from jax import lax
from jax.experimental import pallas as pl
from jax.experimental.pallas import tpu as pltpu
from jax.experimental.pallas import tpu_sc as plsc

NUM_CORES = 2
NUM_SUBCORES = 16
LANES = 16
NW = NUM_CORES * NUM_SUBCORES

NITEMS = 1000001
BATCH = 16384
DIM = 64
NPAN = (NITEMS + 127) // 128
PPW = (NPAN + NW - 1) // NW
ICH = 2048
WCAP = 2 * BATCH + LANES
SEG = 2048
NGRP = 16
GP = (PPW + NGRP - 1) // NGRP

_LANE_SH = 15
_PAN_SH = 22


NBUF = 4


def _sc_body(t_idx_hbm, c_idx_hbm, tt_hbm, stage_hbm,
             idx_v, wpack, gpack, pan0, pan1, pan2, pan3,
             mit, msl, ext, extrow, gb_s,
             psem0, psem1, psem2, psem3, wsem):
    w = lax.axis_index("s") * NUM_CORES + lax.axis_index("c")
    plo = w * PPW
    npan = jnp.minimum(PPW, NPAN - plo)
    lo = plo * 128
    hi = lo + PPW * 128

    def scan_indices(idx_hbm, slot_off, cnt0):
        def chunk(ch, cnt):
            pltpu.sync_copy(idx_hbm.at[pl.ds(ch * ICH, ICH)], idx_v)

            def vec(v, cnt2):
                items = idx_v[pl.ds(v * LANES, LANES)]
                slots = (slot_off + ch * ICH + v * LANES
                         + lax.iota(jnp.int32, LANES))
                m = (items >= lo) & (items < hi)
                packed = (((items - lo) >> 7) << _PAN_SH) \
                    | ((items & 127) << _LANE_SH) | slots
                plsc.store_compressed(wpack.at[pl.ds(cnt2, LANES)], packed,
                                      mask=m)
                return cnt2 + plsc.all_reduce_population_count(m)[0]

            return lax.fori_loop(0, ICH // LANES, vec, cnt)

        return lax.fori_loop(0, BATCH // ICH, chunk, cnt0)

    cnt = scan_indices(t_idx_hbm, 0, 0)
    cnt = scan_indices(c_idx_hbm, BATCH, cnt)
    nv_w = (cnt + LANES - 1) // LANES

    gcnt = 0
    for g in range(NGRP):
        gb_s[g] = gcnt

        def gvec(v, gc, g=g):
            pos = v * LANES + lax.iota(jnp.int32, LANES)
            packed = wpack[pl.ds(v * LANES, LANES)]
            m = (pos < cnt) & ((packed >> (_PAN_SH + 4)) == g)
            plsc.store_compressed(gpack.at[pl.ds(gc, LANES)], packed, mask=m)
            return gc + plsc.all_reduce_population_count(m)[0]

        gcnt = lax.fori_loop(0, nv_w, gvec, gcnt)
    gb_s[NGRP] = gcnt

    pans = (pan0, pan1, pan2, pan3)
    psems = (psem0, psem1, psem2, psem3)

    def fire_panel(lp, buf, sem):
        pltpu.async_copy(tt_hbm.at[:, pl.ds((plo + lp) * 128, 128)], buf, sem)

    def wait_panel(buf, sem):
        pltpu.make_async_copy(tt_hbm.at[:, pl.ds(0, 128)], buf, sem).wait()

    for i in range(NBUF - 1):
        @pl.when(jnp.int32(i) < npan)
        def _(i=i):
            fire_panel(i, pans[i], psems[i])

    def process_panel(lp, buf):
        g0 = gb_s[lp >> 4]
        g1 = gb_s[(lp >> 4) + 1]
        g0a = (g0 // LANES) * LANES

        def segment(s, carry):
            seg_lo = g0a + s * SEG
            seg_hi = jnp.minimum(seg_lo + SEG, g1)

            def match(v, mcnt):
                base = seg_lo + v * LANES
                pos = base + lax.iota(jnp.int32, LANES)
                packed = gpack[pl.ds(base, LANES)]
                m = (pos >= g0) & (pos < seg_hi) & ((packed >> _PAN_SH) == lp)
                plsc.store_compressed(mit.at[pl.ds(mcnt, LANES)],
                                      (packed >> _LANE_SH) & 127, mask=m)
                plsc.store_compressed(msl.at[pl.ds(mcnt, LANES)],
                                      packed & ((1 << _LANE_SH) - 1), mask=m)
                return mcnt + plsc.all_reduce_population_count(m)[0]

            nvec = (seg_hi - seg_lo + LANES - 1) // LANES
            mcnt = lax.fori_loop(0, nvec, match, 0)

            def ext_grp(k, carry2):
                l_vec = mit[pl.ds(k * LANES, LANES)] & 127
                s_vec = msl[pl.ds(k * LANES, LANES)]
                nv = jnp.minimum(LANES, mcnt - k * LANES)
                for d in range(DIM):
                    vals = plsc.load_gather(
                        buf, [jnp.full((LANES,), d, jnp.int32), l_vec])
                    ext[pl.ds(d * LANES, LANES)] = vals
                for j in range(LANES):
                    @pl.when(jnp.int32(j) < nv)
                    def _():
                        for q in range(DIM // LANES):
                            rv = plsc.load_gather(
                                ext,
                                [(q * LANES + lax.iota(jnp.int32, LANES))
                                 * LANES + j])
                            extrow[j, pl.ds(q * LANES, LANES)] = rv
                        pltpu.async_copy(
                            extrow.at[pl.ds(j, 1)],
                            stage_hbm.at[pl.ds(s_vec[j], 1)], wsem)

                def drain(i, c3):
                    pltpu.make_async_copy(
                        extrow.at[pl.ds(0, 1)],
                        stage_hbm.at[pl.ds(0, 1)], wsem).wait()
                    return c3

                lax.fori_loop(0, nv, drain, 0)
                return carry2

            lax.fori_loop(0, (mcnt + LANES - 1) // LANES, ext_grp, 0)
            return carry

        nseg = (g1 - g0a + SEG - 1) // SEG
        lax.fori_loop(0, nseg, segment, 0)

    def ring(g, carry):
        for b in range(NBUF):
            lp = g * NBUF + b

            @pl.when(lp < npan)
            def _(b=b, lp=lp):
                wait_panel(pans[b], psems[b])

                @pl.when(lp + NBUF - 1 < npan)
                def _():
                    nb = (b + NBUF - 1) % NBUF
                    fire_panel(lp + NBUF - 1, pans[nb], psems[nb])

                process_panel(lp, pans[b])

        return carry

    lax.fori_loop(0, (PPW + NBUF - 1) // NBUF, ring, 0)


@functools.cache
def _sc_call():
    return functools.partial(
        pl.kernel,
        out_type=jax.ShapeDtypeStruct((2 * BATCH, DIM), jnp.float32),
        mesh=plsc.VectorSubcoreMesh(
            core_axis_name="c", subcore_axis_name="s",
            num_cores=NUM_CORES, num_subcores=NUM_SUBCORES),
        scratch_types=[
            pltpu.VMEM((ICH,), jnp.int32),
            pltpu.VMEM((WCAP,), jnp.int32),
            pltpu.VMEM((WCAP,), jnp.int32),
            pltpu.VMEM((DIM, 128), jnp.float32),
            pltpu.VMEM((DIM, 128), jnp.float32),
            pltpu.VMEM((DIM, 128), jnp.float32),
            pltpu.VMEM((DIM, 128), jnp.float32),
            pltpu.VMEM((SEG + LANES,), jnp.int32),
            pltpu.VMEM((SEG + LANES,), jnp.int32),
            pltpu.VMEM((DIM * LANES,), jnp.float32),
            pltpu.VMEM((LANES, DIM), jnp.float32),
            pltpu.SMEM((NGRP + 1,), jnp.int32),
            pltpu.SemaphoreType.DMA,
            pltpu.SemaphoreType.DMA,
            pltpu.SemaphoreType.DMA,
            pltpu.SemaphoreType.DMA,
            pltpu.SemaphoreType.DMA,
        ],
        compiler_params=pltpu.CompilerParams(needs_layout_passes=False),
    )(_sc_body)


def _tc_body(t_ref, c_ref, o_ref):
    s = jnp.sum(t_ref[...] * c_ref[...], axis=1)
    o_ref[...] = 1.0 / (1.0 + jnp.exp(-s))


_TC_BLOCK = 512


@functools.cache
def _tc_call():
    grid = BATCH // _TC_BLOCK
    return pl.pallas_call(
        _tc_body,
        grid=(grid,),
        in_specs=[
            pl.BlockSpec((_TC_BLOCK, DIM), lambda i: (i, 0)),
            pl.BlockSpec((_TC_BLOCK, DIM), lambda i: (i + grid, 0)),
        ],
        out_specs=pl.BlockSpec((_TC_BLOCK,), lambda i: (i,)),
        out_shape=jax.ShapeDtypeStruct((BATCH,), jnp.float32),
    )


@jax.jit
def kernel(target_items, context_items, table):
    t = target_items.astype(jnp.int32)
    c = context_items.astype(jnp.int32)
    stage = _sc_call()(t, c, table.T)
    return _tc_call()(stage, stage)


# speedup vs baseline: 1.4485x; 1.4485x over previous
import functools

import jax
import jax.numpy as jnp
from jax import lax
from jax.experimental import pallas as pl
from jax.experimental.pallas import tpu as pltpu
from jax.experimental.pallas import tpu_sc as plsc

NUM_CORES = 2
NUM_SUBCORES = 16
LANES = 16
NW = NUM_CORES * NUM_SUBCORES

NITEMS = 1000001
BATCH = 16384
DIM = 64
NPAN = (NITEMS + 127) // 128
PPW = (NPAN + NW - 1) // NW
ICH = 2048
WCAP = 2 * BATCH + LANES
SEG = 2048
NGRP = 16
GP = (PPW + NGRP - 1) // NGRP

_LANE_SH = 15
_PAN_SH = 22


NBUF = 2
WW = 256


def _sc_body(t_idx_hbm, c_idx_hbm, tt_hbm, stage_hbm,
             idx_v, wpack, gpack, pan0, pan1,
             mit, msl, ext, extrow, gb_s,
             psem0, psem1, wsem):
    w = lax.axis_index("s") * NUM_CORES + lax.axis_index("c")
    plo = w * PPW
    npan = jnp.minimum(PPW, NPAN - plo)
    lo = plo * 128
    hi = lo + PPW * 128

    def scan_indices(idx_hbm, slot_off, cnt0):
        def chunk(ch, cnt):
            pltpu.sync_copy(idx_hbm.at[pl.ds(ch * ICH, ICH)], idx_v)

            def vec(v, cnt2):
                items = idx_v[pl.ds(v * LANES, LANES)]
                slots = (slot_off + ch * ICH + v * LANES
                         + lax.iota(jnp.int32, LANES))
                m = (items >= lo) & (items < hi)
                packed = (((items - lo) >> 7) << _PAN_SH) \
                    | ((items & 127) << _LANE_SH) | slots
                plsc.store_compressed(wpack.at[pl.ds(cnt2, LANES)], packed,
                                      mask=m)
                return cnt2 + plsc.all_reduce_population_count(m)[0]

            return lax.fori_loop(0, ICH // LANES, vec, cnt)

        return lax.fori_loop(0, BATCH // ICH, chunk, cnt0)

    cnt = scan_indices(t_idx_hbm, 0, 0)
    cnt = scan_indices(c_idx_hbm, BATCH, cnt)
    nv_w = (cnt + LANES - 1) // LANES

    gcnt = 0
    for g in range(NGRP):
        gb_s[g] = gcnt

        def gvec(v, gc, g=g):
            pos = v * LANES + lax.iota(jnp.int32, LANES)
            packed = wpack[pl.ds(v * LANES, LANES)]
            m = (pos < cnt) & ((packed >> (_PAN_SH + 4)) == g)
            plsc.store_compressed(gpack.at[pl.ds(gc, LANES)], packed, mask=m)
            return gc + plsc.all_reduce_population_count(m)[0]

        gcnt = lax.fori_loop(0, nv_w, gvec, gcnt)
    gb_s[NGRP] = gcnt

    pans = (pan0, pan1)
    psems = (psem0, psem1)
    nwin = (npan + 1) // 2

    def fire_panel(wd, buf, sem):
        pltpu.async_copy(tt_hbm.at[:, pl.ds((plo + 2 * wd) * 128, WW)],
                         buf, sem)

    def wait_panel(buf, sem):
        pltpu.make_async_copy(tt_hbm.at[:, pl.ds(0, WW)], buf, sem).wait()

    for i in range(NBUF - 1):
        @pl.when(jnp.int32(i) < nwin)
        def _(i=i):
            fire_panel(i, pans[i], psems[i])

    def process_panel(wd, buf):
        lp = wd * 2
        g0 = gb_s[lp >> 4]
        g1 = gb_s[(lp >> 4) + 1]
        g0a = (g0 // LANES) * LANES

        def segment(s, carry):
            seg_lo = g0a + s * SEG
            seg_hi = jnp.minimum(seg_lo + SEG, g1)

            def match(v, mcnt):
                base = seg_lo + v * LANES
                pos = base + lax.iota(jnp.int32, LANES)
                packed = gpack[pl.ds(base, LANES)]
                m = (pos >= g0) & (pos < seg_hi) \
                    & ((packed >> (_PAN_SH + 1)) == wd)
                lane9 = ((packed >> _LANE_SH) & 127) \
                    | (((packed >> _PAN_SH) & 1) << 7)
                plsc.store_compressed(mit.at[pl.ds(mcnt, LANES)],
                                      lane9, mask=m)
                plsc.store_compressed(msl.at[pl.ds(mcnt, LANES)],
                                      packed & ((1 << _LANE_SH) - 1), mask=m)
                return mcnt + plsc.all_reduce_population_count(m)[0]

            nvec = (seg_hi - seg_lo + LANES - 1) // LANES
            mcnt = lax.fori_loop(0, nvec, match, 0)

            def ext_grp(k, carry2):
                l_vec = mit[pl.ds(k * LANES, LANES)] & 255
                s_vec = msl[pl.ds(k * LANES, LANES)]
                nv = jnp.minimum(LANES, mcnt - k * LANES)
                for d in range(DIM):
                    vals = plsc.load_gather(
                        buf, [jnp.full((LANES,), d, jnp.int32), l_vec])
                    ext[pl.ds(d * LANES, LANES)] = vals
                for j in range(LANES):
                    @pl.when(jnp.int32(j) < nv)
                    def _():
                        for q in range(DIM // LANES):
                            rv = plsc.load_gather(
                                ext,
                                [(q * LANES + lax.iota(jnp.int32, LANES))
                                 * LANES + j])
                            extrow[j, pl.ds(q * LANES, LANES)] = rv
                        pltpu.async_copy(
                            extrow.at[pl.ds(j, 1)],
                            stage_hbm.at[pl.ds(s_vec[j], 1)], wsem)

                def drain(i, c3):
                    pltpu.make_async_copy(
                        extrow.at[pl.ds(0, 1)],
                        stage_hbm.at[pl.ds(0, 1)], wsem).wait()
                    return c3

                lax.fori_loop(0, nv, drain, 0)
                return carry2

            lax.fori_loop(0, (mcnt + LANES - 1) // LANES, ext_grp, 0)
            return carry

        nseg = (g1 - g0a + SEG - 1) // SEG
        lax.fori_loop(0, nseg, segment, 0)

    def ring(g, carry):
        for b in range(NBUF):
            wd = g * NBUF + b

            @pl.when(wd < nwin)
            def _(b=b, wd=wd):
                wait_panel(pans[b], psems[b])

                @pl.when(wd + NBUF - 1 < nwin)
                def _():
                    nb = (b + NBUF - 1) % NBUF
                    fire_panel(wd + NBUF - 1, pans[nb], psems[nb])

                process_panel(wd, pans[b])

        return carry

    nwin_max = (PPW + 1) // 2
    lax.fori_loop(0, (nwin_max + NBUF - 1) // NBUF, ring, 0)


@functools.cache
def _sc_call():
    return functools.partial(
        pl.kernel,
        out_type=jax.ShapeDtypeStruct((2 * BATCH, DIM), jnp.float32),
        mesh=plsc.VectorSubcoreMesh(
            core_axis_name="c", subcore_axis_name="s",
            num_cores=NUM_CORES, num_subcores=NUM_SUBCORES),
        scratch_types=[
            pltpu.VMEM((ICH,), jnp.int32),
            pltpu.VMEM((WCAP,), jnp.int32),
            pltpu.VMEM((WCAP,), jnp.int32),
            pltpu.VMEM((DIM, WW), jnp.float32),
            pltpu.VMEM((DIM, WW), jnp.float32),
            pltpu.VMEM((SEG + LANES,), jnp.int32),
            pltpu.VMEM((SEG + LANES,), jnp.int32),
            pltpu.VMEM((DIM * LANES,), jnp.float32),
            pltpu.VMEM((LANES, DIM), jnp.float32),
            pltpu.SMEM((NGRP + 1,), jnp.int32),
            pltpu.SemaphoreType.DMA,
            pltpu.SemaphoreType.DMA,
            pltpu.SemaphoreType.DMA,
        ],
        compiler_params=pltpu.CompilerParams(needs_layout_passes=False),
    )(_sc_body)


def _tc_body(t_ref, c_ref, o_ref):
    s = jnp.sum(t_ref[...] * c_ref[...], axis=1)
    o_ref[...] = 1.0 / (1.0 + jnp.exp(-s))


_TC_BLOCK = 512


@functools.cache
def _tc_call():
    grid = BATCH // _TC_BLOCK
    return pl.pallas_call(
        _tc_body,
        grid=(grid,),
        in_specs=[
            pl.BlockSpec((_TC_BLOCK, DIM), lambda i: (i, 0)),
            pl.BlockSpec((_TC_BLOCK, DIM), lambda i: (i + grid, 0)),
        ],
        out_specs=pl.BlockSpec((_TC_BLOCK,), lambda i: (i,)),
        out_shape=jax.ShapeDtypeStruct((BATCH,), jnp.float32),
    )


@jax.jit
def kernel(target_items, context_items, table):
    t = target_items.astype(jnp.int32)
    c = context_items.astype(jnp.int32)
    stage = _sc_call()(t, c, table.T)
    return _tc_call()(stage, stage)


# speedup vs baseline: 1.6259x; 1.1225x over previous
import functools

import jax
import jax.numpy as jnp
from jax import lax
from jax.experimental import pallas as pl
from jax.experimental.pallas import tpu as pltpu
from jax.experimental.pallas import tpu_sc as plsc

NUM_CORES = 2
NUM_SUBCORES = 16
LANES = 16
NW = NUM_CORES * NUM_SUBCORES

NITEMS = 1000001
BATCH = 16384
DIM = 64
NPAN = (NITEMS + 127) // 128
PPW = (NPAN + NW - 1) // NW
ICH = 2048
WCAP = 2 * BATCH + LANES
SEG = 2048
NGRP = 16
GP = (PPW + NGRP - 1) // NGRP

_LANE_SH = 15
_PAN_SH = 22


NBUF = 2
WW = 384


def _sc_body(t_idx_hbm, c_idx_hbm, tt_hbm, stage_hbm,
             idx_v, wpack, gpack, pan0, pan1,
             mit, msl, ext, extrow, gb_s,
             psem0, psem1, wsem):
    w = lax.axis_index("s") * NUM_CORES + lax.axis_index("c")
    plo = w * PPW
    npan = jnp.minimum(PPW, NPAN - plo)
    lo = plo * 128
    hi = lo + PPW * 128

    def scan_indices(idx_hbm, slot_off, cnt0):
        def chunk(ch, cnt):
            pltpu.sync_copy(idx_hbm.at[pl.ds(ch * ICH, ICH)], idx_v)

            def vec(v, cnt2):
                items = idx_v[pl.ds(v * LANES, LANES)]
                slots = (slot_off + ch * ICH + v * LANES
                         + lax.iota(jnp.int32, LANES))
                m = (items >= lo) & (items < hi)
                packed = (((items - lo) >> 7) << _PAN_SH) \
                    | ((items & 127) << _LANE_SH) | slots
                plsc.store_compressed(wpack.at[pl.ds(cnt2, LANES)], packed,
                                      mask=m)
                return cnt2 + plsc.all_reduce_population_count(m)[0]

            return lax.fori_loop(0, ICH // LANES, vec, cnt)

        return lax.fori_loop(0, BATCH // ICH, chunk, cnt0)

    cnt = scan_indices(t_idx_hbm, 0, 0)
    cnt = scan_indices(c_idx_hbm, BATCH, cnt)
    nv_w = (cnt + LANES - 1) // LANES

    gcnt = 0
    for g in range(NGRP):
        gb_s[g] = gcnt

        def gvec(v, gc, g=g):
            pos = v * LANES + lax.iota(jnp.int32, LANES)
            packed = wpack[pl.ds(v * LANES, LANES)]
            m = (pos < cnt) & ((packed >> (_PAN_SH + 4)) == g)
            plsc.store_compressed(gpack.at[pl.ds(gc, LANES)], packed, mask=m)
            return gc + plsc.all_reduce_population_count(m)[0]

        gcnt = lax.fori_loop(0, nv_w, gvec, gcnt)
    gb_s[NGRP] = gcnt

    pans = (pan0, pan1)
    psems = (psem0, psem1)
    nwin = (npan + 2) // 3

    def win_pan0(wd):
        return jnp.minimum(plo + 3 * wd, NPAN - 3)

    def fire_panel(wd, buf, sem):
        pltpu.async_copy(tt_hbm.at[:, pl.ds(win_pan0(wd) * 128, WW)],
                         buf, sem)

    def wait_panel(buf, sem):
        pltpu.make_async_copy(tt_hbm.at[:, pl.ds(0, WW)], buf, sem).wait()

    for i in range(NBUF - 1):
        @pl.when(jnp.int32(i) < nwin)
        def _(i=i):
            fire_panel(i, pans[i], psems[i])

    def process_panel(wd, buf):
        pan0_w = win_pan0(wd)
        g0 = gb_s[(3 * wd) >> 4]
        g1 = gb_s[((3 * wd + 2) >> 4) + 1]
        g0a = (g0 // LANES) * LANES

        def segment(s, carry):
            seg_lo = g0a + s * SEG
            seg_hi = jnp.minimum(seg_lo + SEG, g1)

            def match(v, mcnt):
                base = seg_lo + v * LANES
                pos = base + lax.iota(jnp.int32, LANES)
                packed = gpack[pl.ds(base, LANES)]
                gpan = (packed >> _PAN_SH) + plo
                m = (pos >= g0) & (pos < seg_hi) \
                    & (gpan >= pan0_w) & (gpan < pan0_w + 3)
                lane_w = ((gpan - pan0_w) << 7) \
                    | ((packed >> _LANE_SH) & 127)
                plsc.store_compressed(mit.at[pl.ds(mcnt, LANES)],
                                      lane_w, mask=m)
                plsc.store_compressed(msl.at[pl.ds(mcnt, LANES)],
                                      packed & ((1 << _LANE_SH) - 1), mask=m)
                return mcnt + plsc.all_reduce_population_count(m)[0]

            nvec = (seg_hi - seg_lo + LANES - 1) // LANES
            mcnt = lax.fori_loop(0, nvec, match, 0)

            def ext_grp(k, carry2):
                l_vec = jnp.minimum(mit[pl.ds(k * LANES, LANES)] & 511,
                                    WW - 1)
                s_vec = msl[pl.ds(k * LANES, LANES)]
                nv = jnp.minimum(LANES, mcnt - k * LANES)
                for d in range(DIM):
                    vals = plsc.load_gather(
                        buf, [jnp.full((LANES,), d, jnp.int32), l_vec])
                    ext[pl.ds(d * LANES, LANES)] = vals
                for j in range(LANES):
                    @pl.when(jnp.int32(j) < nv)
                    def _():
                        for q in range(DIM // LANES):
                            rv = plsc.load_gather(
                                ext,
                                [(q * LANES + lax.iota(jnp.int32, LANES))
                                 * LANES + j])
                            extrow[j, pl.ds(q * LANES, LANES)] = rv
                        pltpu.async_copy(
                            extrow.at[pl.ds(j, 1)],
                            stage_hbm.at[pl.ds(s_vec[j], 1)], wsem)

                def drain(i, c3):
                    pltpu.make_async_copy(
                        extrow.at[pl.ds(0, 1)],
                        stage_hbm.at[pl.ds(0, 1)], wsem).wait()
                    return c3

                lax.fori_loop(0, nv, drain, 0)
                return carry2

            lax.fori_loop(0, (mcnt + LANES - 1) // LANES, ext_grp, 0)
            return carry

        nseg = (g1 - g0a + SEG - 1) // SEG
        lax.fori_loop(0, nseg, segment, 0)

    def ring(g, carry):
        for b in range(NBUF):
            wd = g * NBUF + b

            @pl.when(wd < nwin)
            def _(b=b, wd=wd):
                wait_panel(pans[b], psems[b])

                @pl.when(wd + NBUF - 1 < nwin)
                def _():
                    nb = (b + NBUF - 1) % NBUF
                    fire_panel(wd + NBUF - 1, pans[nb], psems[nb])

                process_panel(wd, pans[b])

        return carry

    nwin_max = (PPW + 2) // 3
    lax.fori_loop(0, (nwin_max + NBUF - 1) // NBUF, ring, 0)


@functools.cache
def _sc_call():
    return functools.partial(
        pl.kernel,
        out_type=jax.ShapeDtypeStruct((2 * BATCH, DIM), jnp.float32),
        mesh=plsc.VectorSubcoreMesh(
            core_axis_name="c", subcore_axis_name="s",
            num_cores=NUM_CORES, num_subcores=NUM_SUBCORES),
        scratch_types=[
            pltpu.VMEM((ICH,), jnp.int32),
            pltpu.VMEM((WCAP,), jnp.int32),
            pltpu.VMEM((WCAP,), jnp.int32),
            pltpu.VMEM((DIM, WW), jnp.float32),
            pltpu.VMEM((DIM, WW), jnp.float32),
            pltpu.VMEM((SEG + LANES,), jnp.int32),
            pltpu.VMEM((SEG + LANES,), jnp.int32),
            pltpu.VMEM((DIM * LANES,), jnp.float32),
            pltpu.VMEM((LANES, DIM), jnp.float32),
            pltpu.SMEM((NGRP + 1,), jnp.int32),
            pltpu.SemaphoreType.DMA,
            pltpu.SemaphoreType.DMA,
            pltpu.SemaphoreType.DMA,
        ],
        compiler_params=pltpu.CompilerParams(needs_layout_passes=False),
    )(_sc_body)


def _tc_body(t_ref, c_ref, o_ref):
    s = jnp.sum(t_ref[...] * c_ref[...], axis=1)
    o_ref[...] = 1.0 / (1.0 + jnp.exp(-s))


_TC_BLOCK = 512


@functools.cache
def _tc_call():
    grid = BATCH // _TC_BLOCK
    return pl.pallas_call(
        _tc_body,
        grid=(grid,),
        in_specs=[
            pl.BlockSpec((_TC_BLOCK, DIM), lambda i: (i, 0)),
            pl.BlockSpec((_TC_BLOCK, DIM), lambda i: (i + grid, 0)),
        ],
        out_specs=pl.BlockSpec((_TC_BLOCK,), lambda i: (i,)),
        out_shape=jax.ShapeDtypeStruct((BATCH,), jnp.float32),
    )


@jax.jit
def kernel(target_items, context_items, table):
    t = target_items.astype(jnp.int32)
    c = context_items.astype(jnp.int32)
    stage = _sc_call()(t, c, table.T)
    return _tc_call()(stage, stage)
